# baseline (speedup 1.0000x reference)
import functools

import jax
import jax.numpy as jnp
from jax import lax
from jax.experimental import pallas as pl
from jax.experimental.pallas import tpu as pltpu
from jax.experimental.pallas import tpu_sc as plsc

VOCAB = 1000
BATCH = 4096
SEQ = 20
B = BATCH * SEQ
NW = 32
BPW = B // NW
K = 40
CH = BPW // K

_mesh = plsc.VectorSubcoreMesh(core_axis_name="c", subcore_axis_name="s")


@functools.partial(
    pl.kernel,
    mesh=_mesh,
    compiler_params=pltpu.CompilerParams(use_tc_tiling_on_sc=False),
    out_type=jax.ShapeDtypeStruct((B, VOCAB), jnp.float32),
    scratch_types=[
        pltpu.VMEM((BPW,), jnp.int32),
        pltpu.VMEM((K, VOCAB), jnp.float32),
        pltpu.VMEM((K, VOCAB), jnp.float32),
        pltpu.SemaphoreType.DMA,
        pltpu.SemaphoreType.DMA,
        pltpu.SemaphoreType.DMA,
        pltpu.SemaphoreType.DMA,
    ],
)
def _gather_kernel(table_hbm, idx_hbm, out_hbm, idx_v, buf0, buf1,
                   gsem0, gsem1, ssem0, ssem1):
    wid = lax.axis_index("s") * 2 + lax.axis_index("c")
    base = wid * BPW
    pltpu.sync_copy(idx_hbm.at[pl.ds(base, BPW)], idx_v)

    bufs = (buf0, buf1)
    gsems = (gsem0, gsem1)
    ssems = (ssem0, ssem1)

    def gstart(b, j):
        pltpu.async_copy(
            table_hbm.at[idx_v.at[pl.ds(j * K, K)]], bufs[b], gsems[b]
        )

    def gwait(b):
        pltpu.make_async_copy(
            table_hbm.at[idx_v.at[pl.ds(0, K)]], bufs[b], gsems[b]
        ).wait()

    def sstart(b, j):
        pltpu.async_copy(bufs[b], out_hbm.at[pl.ds(base + j * K, K)], ssems[b])

    def swait(b):
        pltpu.make_async_copy(
            bufs[b], out_hbm.at[pl.ds(base, K)], ssems[b]
        ).wait()

    gstart(0, 0)
    gstart(1, 1)

    def pair(p, carry):
        gwait(0)
        gstart(0, 2 * p + 2)
        gwait(1)
        gstart(1, 2 * p + 3)
        return carry

    lax.fori_loop(0, CH // 2 - 1, pair, 0)
    gwait(0)
    gwait(1)
    sstart(0, 0)
    swait(0)


def kernel(x_ids, logits_table):
    idx = x_ids.reshape(-1).astype(jnp.int32)
    out = _gather_kernel(logits_table, idx)
    return out.reshape(x_ids.shape + (VOCAB,))


# speedup vs baseline: 1.0065x; 1.0065x over previous
import functools

import jax
import jax.numpy as jnp
from jax import lax
from jax.experimental import pallas as pl
from jax.experimental.pallas import tpu as pltpu
from jax.experimental.pallas import tpu_sc as plsc

VOCAB = 1000
BATCH = 4096
SEQ = 20
B = BATCH * SEQ
NW = 32
BPW = B // NW
K = 32
CH = BPW // K

_mesh = plsc.VectorSubcoreMesh(core_axis_name="c", subcore_axis_name="s")


@functools.partial(
    pl.kernel,
    mesh=_mesh,
    compiler_params=pltpu.CompilerParams(use_tc_tiling_on_sc=False),
    out_type=jax.ShapeDtypeStruct((B, VOCAB), jnp.float32),
    scratch_types=[
        pltpu.VMEM((BPW,), jnp.int32),
        pltpu.VMEM((K, VOCAB), jnp.float32),
        pltpu.VMEM((K, VOCAB), jnp.float32),
        pltpu.VMEM_SHARED((VOCAB, VOCAB), jnp.float32),
        pltpu.SemaphoreType.DMA,
        pltpu.SemaphoreType.DMA,
        pltpu.SemaphoreType.DMA,
        pltpu.SemaphoreType.DMA,
    ],
)
def _gather_kernel(table_hbm, idx_hbm, out_hbm, idx_v, buf0, buf1, table_sp,
                   gsem0, gsem1, ssem0, ssem1):
    cid = lax.axis_index("c")
    sid = lax.axis_index("s")
    wid = sid * 2 + cid
    base = wid * BPW
    pltpu.sync_copy(idx_hbm.at[pl.ds(base, BPW)], idx_v)

    @pl.when(sid < 10)
    def _load_table():
        pltpu.sync_copy(
            table_hbm.at[pl.ds(sid * 100, 100)],
            table_sp.at[pl.ds(sid * 100, 100)],
        )

    plsc.subcore_barrier()

    bufs = (buf0, buf1)
    gsems = (gsem0, gsem1)
    ssems = (ssem0, ssem1)

    def gstart(b, j):
        pltpu.async_copy(
            table_sp.at[idx_v.at[pl.ds(j * K, K)]], bufs[b], gsems[b]
        )

    def gwait(b):
        pltpu.make_async_copy(
            table_sp.at[idx_v.at[pl.ds(0, K)]], bufs[b], gsems[b]
        ).wait()

    def sstart(b, j):
        pltpu.async_copy(bufs[b], out_hbm.at[pl.ds(base + j * K, K)], ssems[b])

    def swait(b):
        pltpu.make_async_copy(
            bufs[b], out_hbm.at[pl.ds(base, K)], ssems[b]
        ).wait()

    gstart(0, 0)
    gwait(0)
    sstart(0, 0)
    gstart(1, 1)

    def pair(p, carry):
        j = 2 * p + 1
        gwait(1)
        sstart(1, j)
        swait(0)
        gstart(0, j + 1)
        gwait(0)
        sstart(0, j + 1)
        swait(1)
        gstart(1, j + 2)
        return carry

    lax.fori_loop(0, CH // 2 - 1, pair, 0)

    gwait(1)
    sstart(1, CH - 1)
    swait(0)
    swait(1)


def kernel(x_ids, logits_table):
    idx = x_ids.reshape(-1).astype(jnp.int32)
    out = _gather_kernel(logits_table, idx)
    return out.reshape(x_ids.shape + (VOCAB,))
